# baseline (speedup 1.0000x reference)
import math

import jax
import jax.numpy as jnp
from jax.experimental import pallas as pl
from jax.experimental.pallas import tpu as pltpu

_N_DAYS_YEAR = 365
_N_TIME_MODELS = 12
_N_STEP_MODELS = 4
_N_STEPS = 48
_TIME_SPAN = math.ceil(_N_DAYS_YEAR / _N_TIME_MODELS)
_STEP_SPAN = math.ceil(_N_STEPS / _N_STEP_MODELS)

_S_BLK = 2000


def _body(gs_ref, od_ref, feat_ref, coef_ref, bias_ref, out_ref):
    del gs_ref, od_ref
    f = feat_ref[0, 0]
    c = coef_ref[0]
    b = bias_ref[0]
    col_f = jax.lax.broadcasted_iota(jnp.int32, (8, 32), 1) // 4
    row_f = jax.lax.broadcasted_iota(jnp.int32, (8, 32), 0)
    expand = (col_f == row_f).astype(jnp.float32)
    col_vp = jax.lax.broadcasted_iota(jnp.int32, (32, 4), 0) % 4
    row_vp = jax.lax.broadcasted_iota(jnp.int32, (32, 4), 1)
    reduce = (col_vp == row_vp).astype(jnp.float32)
    fe = jnp.dot(f, expand, preferred_element_type=jnp.float32)
    out = jnp.dot(c * fe, reduce, preferred_element_type=jnp.float32) + b
    out_ref[0] = out


def kernel(day_of_year, step_idx, features, coefs, biases):
    n_time, n_step, n_stations, in_f, n_var, n_par = coefs.shape
    batch = features.shape[0]
    n_groups = n_time * n_step
    vp = n_var * n_par

    g = (day_of_year // _TIME_SPAN).astype(jnp.int32) * n_step + (
        step_idx // _STEP_SPAN
    ).astype(jnp.int32)
    order = jnp.argsort(g).astype(jnp.int32)
    g_sorted = jnp.take(g, order)

    coefs_r = coefs.reshape(n_groups, n_stations, in_f * vp)
    biases_r = biases.reshape(n_groups, n_stations, vp)

    sb = n_stations // _S_BLK

    grid_spec = pltpu.PrefetchScalarGridSpec(
        num_scalar_prefetch=2,
        grid=(sb, batch),
        in_specs=[
            pl.BlockSpec(
                (1, 1, _S_BLK, in_f),
                lambda s, b, gs, od: (od[b], 0, s, 0),
            ),
            pl.BlockSpec(
                (1, _S_BLK, in_f * vp),
                lambda s, b, gs, od: (gs[b], s, 0),
            ),
            pl.BlockSpec(
                (1, _S_BLK, vp),
                lambda s, b, gs, od: (gs[b], s, 0),
            ),
        ],
        out_specs=pl.BlockSpec(
            (1, _S_BLK, vp),
            lambda s, b, gs, od: (od[b], s, 0),
        ),
    )

    out = pl.pallas_call(
        _body,
        grid_spec=grid_spec,
        out_shape=jax.ShapeDtypeStruct((batch, n_stations, vp), jnp.float32),
        compiler_params=pltpu.CompilerParams(
            dimension_semantics=("arbitrary", "arbitrary"),
        ),
    )(g_sorted, order, features, coefs_r, biases_r)

    return out.reshape(batch, n_stations, n_var, n_par)
